# baseline (speedup 1.0000x reference)
import jax
import jax.numpy as jnp
from jax import lax
from jax.experimental import pallas as pl
from jax.experimental.pallas import tpu as pltpu
from jax.experimental.pallas import tpu_sc as plsc

_B = 4096
_D0, _D1, _D2 = 128, 64, 32
_NC, _NS, _L = 2, 16, 16
_NW = _NC * _NS
_BPW = _B // _NW


def _sc_body(idx_hbm, t0_hbm, t1_hbm, t2_hbm, out_hbm,
             idx_v, b0, b1, b2, sem0, sem1, sem2):
    wid = lax.axis_index("s") * _NC + lax.axis_index("c")
    base = wid * _BPW
    pltpu.sync_copy(idx_hbm.at[pl.ds(base, _BPW)], idx_v)
    c0 = pltpu.async_copy(t0_hbm.at[idx_v], b0, sem0)
    c1 = pltpu.async_copy(t1_hbm.at[idx_v], b1, sem1)
    c2 = pltpu.async_copy(t2_hbm.at[idx_v], b2, sem2)
    c2.wait()
    c1.wait()
    c0.wait()
    third = jnp.float32(1.0 / 3.0)

    def row(r, carry):
        for j in range(_D0 // _L):
            v = b0[r, pl.ds(_L * j, _L)]
            if _L * j < _D1:
                v = v + b1[r, pl.ds(_L * j, _L)]
            if _L * j < _D2:
                v = v + b2[r, pl.ds(_L * j, _L)]
            b0[r, pl.ds(_L * j, _L)] = v * third
        return carry

    lax.fori_loop(0, _BPW, row, 0)
    pltpu.sync_copy(b0, out_hbm.at[pl.ds(base, _BPW)])


def kernel(indexes, table0, table1, table2):
    mesh = plsc.VectorSubcoreMesh(core_axis_name="c", subcore_axis_name="s")
    k = pl.kernel(
        _sc_body,
        out_type=jax.ShapeDtypeStruct((_B, _D0), jnp.float32),
        mesh=mesh,
        compiler_params=pltpu.CompilerParams(use_tc_tiling_on_sc=False),
        scratch_types=[
            pltpu.VMEM((_BPW,), jnp.int32),
            pltpu.VMEM((_BPW, _D0), jnp.float32),
            pltpu.VMEM((_BPW, _D1), jnp.float32),
            pltpu.VMEM((_BPW, _D2), jnp.float32),
            pltpu.SemaphoreType.DMA,
            pltpu.SemaphoreType.DMA,
            pltpu.SemaphoreType.DMA,
        ],
    )
    return k(indexes.astype(jnp.int32), table0, table1, table2)
